# baseline (speedup 1.0000x reference)
import functools
import math

import jax
import jax.numpy as jnp
from jax import lax
from jax.experimental import pallas as pl
from jax.experimental.pallas import tpu as pltpu
from jax.experimental.pallas import tpu_sc as plsc

NC = 2
NS = 16
NW = NC * NS
CH = 128
KB = 8
ZB = 128
LANE = 16
BN = 2048


def _sc_mesh():
    return plsc.VectorSubcoreMesh(
        core_axis_name="c", subcore_axis_name="s", num_cores=NC, num_subcores=NS)


def _sc_segment_pass(table, idxg, idxs, n_pad, split_by_core, core_shift=0):
    nch = idxs.shape[0] - KB
    per_unit = nch // (NW if split_by_core else NS)
    nblk = per_unit // KB
    assert per_unit * (NW if split_by_core else NS) == nch and nblk * KB == per_unit
    nacc = ((n_pad + LANE + NS * 8 - 1) // (NS * 8)) * NS * 8
    rpt_zero = nacc // NS
    rpt_out = n_pad // NS
    assert rpt_out * NS == n_pad and rpt_out % 8 == 0 and rpt_zero % 8 == 0
    nz_full, nz_rem = rpt_zero // ZB, rpt_zero % ZB

    @functools.partial(
        pl.kernel,
        out_type=jax.ShapeDtypeStruct((NC, n_pad, LANE), jnp.float32),
        mesh=_sc_mesh(),
        scratch_types=[
            pltpu.VMEM((2, KB, CH), jnp.int32),
            pltpu.VMEM((2, KB, CH), jnp.int32),
            pltpu.VMEM((2, KB // 2, CH, LANE), jnp.float32),
            pltpu.VMEM((ZB, LANE), jnp.float32),
            pltpu.VMEM((CH,), jnp.int32),
            pltpu.VMEM_SHARED((nacc, LANE), jnp.float32),
            pltpu.SemaphoreType.DMA,
            pltpu.SemaphoreType.DMA,
            pltpu.SemaphoreType.DMA,
            pltpu.SemaphoreType.DMA,
        ],
        compiler_params=pltpu.CompilerParams(use_tc_tiling_on_sc=False),
    )
    def body(table_h, idxg_h, idxs_h, out_h,
             gidx_v, sidx_v, rows_v, zero_v, trash_v, acc_s,
             isem, gsem, ssem0, ssem1):
        ssems = (ssem0, ssem1)
        SUB = KB // 2
        c = lax.axis_index("c")
        s = lax.axis_index("s")

        def _fill_zero(i, carry):
            zero_v[i, :] = jnp.zeros((LANE,), jnp.float32)
            return carry
        lax.fori_loop(0, ZB, _fill_zero, 0)

        zbase = s * rpt_zero
        for i in range(nz_full):
            pltpu.sync_copy(zero_v, acc_s.at[pl.ds(zbase + i * ZB, ZB)])
        if nz_rem:
            pltpu.sync_copy(zero_v.at[pl.ds(0, nz_rem)],
                            acc_s.at[pl.ds(zbase + nz_full * ZB, nz_rem)])
        plsc.subcore_barrier()

        unit = s * NC + c if split_by_core else s
        first = unit * per_unit

        def issue_idx_load(b, p):
            ck0 = first + b * KB
            pltpu.async_copy(idxg_h.at[pl.ds(ck0, KB)], gidx_v.at[p], isem)
            pltpu.async_copy(idxs_h.at[pl.ds(ck0, KB)], sidx_v.at[p], isem)

        def wait_idx(p):
            pltpu.make_async_copy(idxg_h.at[pl.ds(0, KB)], gidx_v.at[p], isem).wait()
            pltpu.make_async_copy(idxs_h.at[pl.ds(0, KB)], sidx_v.at[p], isem).wait()

        def drain_adds(q):
            for j in range(SUB):
                pltpu.make_async_copy(table_h.at[pl.ds(0, CH)],
                                      rows_v.at[q].at[j], ssems[q]).wait()

        def emit_block(b, p):
            wait_idx(p)
            issue_idx_load(b + 1, 1 - p)
            if core_shift:
                @pl.when(c == 1)
                def _():
                    off = jnp.full((LANE,), core_shift, jnp.int32)
                    for k in range(KB):
                        for g in range(CH // LANE):
                            sl = pl.ds(g * LANE, LANE)
                            gidx_v[p, k, sl] = gidx_v[p, k, sl] + off
            gathers = []
            for q in range(2):
                drain_adds(q)
                gathers.append([
                    pltpu.async_copy(table_h.at[gidx_v.at[p].at[q * SUB + j]],
                                     rows_v.at[q].at[j], gsem)
                    for j in range(SUB)
                ])
            for q in range(2):
                for j in range(SUB):
                    gathers[q][j].wait()
                    pltpu.async_copy(rows_v.at[q].at[j],
                                     acc_s.at[sidx_v.at[p].at[q * SUB + j]],
                                     ssems[q], add=True)

        for g in range(CH // LANE):
            trash_v[pl.ds(g * LANE, LANE)] = jnp.full((LANE,), n_pad, jnp.int32)
        for q in range(2):
            for _ in range(SUB):
                pltpu.async_copy(zero_v, acc_s.at[trash_v], ssems[q], add=True)

        issue_idx_load(0, 0)

        def pair(i, carry):
            emit_block(2 * i, 0)
            emit_block(2 * i + 1, 1)
            return carry
        lax.fori_loop(0, nblk // 2, pair, 0)
        if nblk % 2:
            emit_block(nblk - 1, 0)
        wait_idx(nblk % 2)
        for q in range(2):
            drain_adds(q)
        plsc.subcore_barrier()

        obase = s * rpt_out
        pltpu.sync_copy(acc_s.at[pl.ds(obase, rpt_out)],
                        out_h.at[c].at[pl.ds(obase, rpt_out)])

    return body(table, idxg, idxs)


def _combine_block(s_r, d_r, h_prev, bc_r, brp_r, h, r_num, bn):
    s0 = s_r[0]
    s1 = s_r[1]
    d = d_r[0] + d_r[1]
    hb = jnp.dot(h_prev, bc_r, preferred_element_type=jnp.float32)
    acc = jnp.concatenate([s0, s1], axis=-1)
    acc = acc + jnp.dot(d, brp_r, preferred_element_type=jnp.float32)
    for r in range(r_num):
        dr = d[:, r:r + 1]
        acc = acc + dr * hb[:, r * h:(r + 1) * h]
    return acc


def _tc_stage1(x, W0, b0, A1P):
    n, din = x.shape
    h = W0.shape[1]
    wide = A1P.shape[1]

    def body(x_r, w0_r, b0_r, ap_r, emb_r, g_r):
        e = jnp.dot(x_r[...], w0_r[...], preferred_element_type=jnp.float32)
        e = e + b0_r[...]
        emb_r[...] = e
        g_r[...] = jnp.dot(e, ap_r[...], preferred_element_type=jnp.float32)

    return pl.pallas_call(
        body,
        grid=(pl.cdiv(n, BN),),
        in_specs=[
            pl.BlockSpec((BN, din), lambda i: (i, 0)),
            pl.BlockSpec((din, h), lambda i: (0, 0)),
            pl.BlockSpec((1, h), lambda i: (0, 0)),
            pl.BlockSpec((h, wide), lambda i: (0, 0)),
        ],
        out_specs=[
            pl.BlockSpec((BN, h), lambda i: (i, 0)),
            pl.BlockSpec((BN, wide), lambda i: (i, 0)),
        ],
        out_shape=[
            jax.ShapeDtypeStruct((n, h), jnp.float32),
            jax.ShapeDtypeStruct((n, wide), jnp.float32),
        ],
    )(x, W0, b0.reshape(1, h), A1P)


def _tc_stage2(s1, deg, emb0, brp1, B1C, A2P):
    n = emb0.shape[0]
    h = B1C.shape[0]
    r_num = B1C.shape[1] // h
    wide = A2P.shape[1]

    def body(s_r, d_r, e0_r, br_r, bc_r, ap_r, g_r, e1_r):
        emb = _combine_block(s_r, d_r, e0_r[...], bc_r[...], br_r[...], h,
                             r_num, BN)
        e1_r[...] = emb
        g_r[...] = jnp.dot(emb, ap_r[...], preferred_element_type=jnp.float32)

    return pl.pallas_call(
        body,
        grid=(pl.cdiv(n, BN),),
        in_specs=[
            pl.BlockSpec((NC, BN, LANE), lambda i: (0, i, 0)),
            pl.BlockSpec((NC, BN, LANE), lambda i: (0, i, 0)),
            pl.BlockSpec((BN, h), lambda i: (i, 0)),
            pl.BlockSpec((LANE, h), lambda i: (0, 0)),
            pl.BlockSpec((h, wide), lambda i: (0, 0)),
            pl.BlockSpec((h, wide), lambda i: (0, 0)),
        ],
        out_specs=[
            pl.BlockSpec((BN, wide), lambda i: (i, 0)),
            pl.BlockSpec((BN, h), lambda i: (i, 0)),
        ],
        out_shape=[
            jax.ShapeDtypeStruct((n, wide), jnp.float32),
            jax.ShapeDtypeStruct((n, h), jnp.float32),
        ],
    )(s1, deg, emb0, brp1, B1C, A2P)


def _tc_stage3(s2, deg, emb1, brp2, B2C, emb0, Wf1, bf1, Wf2, bf2):
    n = emb0.shape[0]
    h = B2C.shape[0]
    r_num = B2C.shape[1] // h
    dmid = Wf1.shape[1]
    dout = Wf2.shape[1]

    def body(s_r, d_r, e1_r, br_r, bc_r, e0_r, wf1_r, bf1_r, wf2_r, bf2_r, o_r):
        emb2 = _combine_block(s_r, d_r, e1_r[...], bc_r[...], br_r[...], h,
                              r_num, BN)
        z = jnp.concatenate([emb2, e0_r[...]], axis=-1)
        z1 = jnp.dot(z, wf1_r[...], preferred_element_type=jnp.float32)
        z1 = z1 + bf1_r[...]
        z1 = jnp.where(z1 >= 0, z1, 0.01 * z1)
        o = jnp.dot(z1, wf2_r[...], preferred_element_type=jnp.float32)
        o = o + bf2_r[...]
        o_r[...] = jnp.where(o >= 0, o, 0.01 * o)

    return pl.pallas_call(
        body,
        grid=(pl.cdiv(n, BN),),
        in_specs=[
            pl.BlockSpec((NC, BN, LANE), lambda i: (0, i, 0)),
            pl.BlockSpec((NC, BN, LANE), lambda i: (0, i, 0)),
            pl.BlockSpec((BN, h), lambda i: (i, 0)),
            pl.BlockSpec((LANE, h), lambda i: (0, 0)),
            pl.BlockSpec((h, r_num * h), lambda i: (0, 0)),
            pl.BlockSpec((BN, h), lambda i: (i, 0)),
            pl.BlockSpec((2 * h, dmid), lambda i: (0, 0)),
            pl.BlockSpec((1, dmid), lambda i: (0, 0)),
            pl.BlockSpec((dmid, dout), lambda i: (0, 0)),
            pl.BlockSpec((1, dout), lambda i: (0, 0)),
        ],
        out_specs=[pl.BlockSpec((BN, dout), lambda i: (i, 0))],
        out_shape=[jax.ShapeDtypeStruct((n, dout), jnp.float32)],
    )(s2, deg, emb1, brp2, B2C, emb0, Wf1, bf1.reshape(1, dmid), Wf2,
      bf2.reshape(1, dout))[0]


def _a_perm(Wr, h):
    r_num = Wr.shape[0]
    a = Wr[:, :h, :]
    cols = [a[t][:, c * LANE:(c + 1) * LANE]
            for c in range(NC) for t in range(r_num)]
    return jnp.concatenate(cols, axis=1)


def _b_cat(Wr, h):
    b = Wr[:, h:, :]
    return jnp.concatenate([b[t] for t in range(Wr.shape[0])], axis=1)


def kernel(x, edge_index, edge_type, edge_time, pos,
           W0, b0, Wr1, br1, Wr2, br2, Wf1, bf1, Wf2, bf2):
    del edge_time, pos
    n = x.shape[0]
    h = W0.shape[1]
    r_num = Wr1.shape[0]
    e_num = edge_index.shape[1]
    assert h == 2 * LANE

    row = edge_index[0]
    col = edge_index[1]

    n_pad = ((n + NS * 8 - 1) // (NS * 8)) * NS * 8
    align = NW * KB
    nch = math.ceil(math.ceil(e_num / CH) / align) * align
    padn = nch * CH - e_num
    padn_a = padn + KB * CH
    gbase = row * (NC * r_num) + edge_type
    idxg = jnp.pad(gbase, (0, padn_a)).reshape(nch + KB, CH)
    idxs = jnp.pad(col, (0, padn_a), constant_values=n_pad).reshape(nch + KB, CH)
    rep = 4096
    idxt = jnp.pad(edge_type, (0, padn_a))
    idxt = idxt + 2 * r_num * (jnp.arange(idxt.shape[0], dtype=jnp.int32) & (rep - 1))
    idxt = idxt.reshape(nch + KB, CH)
    onehot = jnp.tile(jnp.eye(2 * r_num, LANE, dtype=jnp.float32), (rep, 1))

    A1P, B1C = _a_perm(Wr1, h), _b_cat(Wr1, h)
    A2P, B2C = _a_perm(Wr2, h), _b_cat(Wr2, h)

    brp1 = jnp.pad(br1, ((0, LANE - r_num), (0, 0)))
    brp2 = jnp.pad(br2, ((0, LANE - r_num), (0, 0)))

    deg = _sc_segment_pass(onehot, idxt, idxs, n_pad, split_by_core=True)
    emb0, g1 = _tc_stage1(x, W0, b0, A1P)
    s1 = _sc_segment_pass(g1.reshape(n * NC * r_num, LANE), idxg, idxs, n_pad,
                          split_by_core=False, core_shift=r_num)
    g2, emb1 = _tc_stage2(s1, deg, emb0, brp1, B1C, A2P)
    s2 = _sc_segment_pass(g2.reshape(n * NC * r_num, LANE), idxg, idxs, n_pad,
                          split_by_core=False, core_shift=r_num)
    return _tc_stage3(s2, deg, emb1, brp2, B2C, emb0, Wf1, bf1,
                      Wf2, bf2)


# speedup vs baseline: 1.0039x; 1.0039x over previous
import functools
import math

import jax
import jax.numpy as jnp
from jax import lax
from jax.experimental import pallas as pl
from jax.experimental.pallas import tpu as pltpu
from jax.experimental.pallas import tpu_sc as plsc

NC = 2
NS = 16
NW = NC * NS
CH = 128
KB = 8
ZB = 128
LANE = 16
BN = 2048


def _sc_mesh():
    return plsc.VectorSubcoreMesh(
        core_axis_name="c", subcore_axis_name="s", num_cores=NC, num_subcores=NS)


def _sc_segment_pass(table, idxg, idxs, n_pad, split_by_core, core_shift=0):
    nch = idxs.shape[0] - KB
    per_unit = nch // (NW if split_by_core else NS)
    nblk = per_unit // KB
    assert per_unit * (NW if split_by_core else NS) == nch and nblk * KB == per_unit
    nacc = ((n_pad + LANE + NS * 8 - 1) // (NS * 8)) * NS * 8
    rpt_zero = nacc // NS
    rpt_out = n_pad // NS
    assert rpt_out * NS == n_pad and rpt_out % 8 == 0 and rpt_zero % 8 == 0
    nz_full, nz_rem = rpt_zero // ZB, rpt_zero % ZB

    @functools.partial(
        pl.kernel,
        out_type=jax.ShapeDtypeStruct((NC, n_pad, LANE), jnp.float32),
        mesh=_sc_mesh(),
        scratch_types=[
            pltpu.VMEM((2, KB, CH), jnp.int32),
            pltpu.VMEM((2, KB, CH), jnp.int32),
            pltpu.VMEM((KB, CH, LANE), jnp.float32),
            pltpu.VMEM((ZB, LANE), jnp.float32),
            pltpu.VMEM_SHARED((nacc, LANE), jnp.float32),
            pltpu.SemaphoreType.DMA,
            pltpu.SemaphoreType.DMA,
            pltpu.SemaphoreType.DMA,
        ],
        compiler_params=pltpu.CompilerParams(use_tc_tiling_on_sc=False),
    )
    def body(table_h, idxg_h, idxs_h, out_h,
             gidx_v, sidx_v, rows_v, zero_v, acc_s, isem, gsem, ssem):
        c = lax.axis_index("c")
        s = lax.axis_index("s")

        def _fill_zero(i, carry):
            zero_v[i, :] = jnp.zeros((LANE,), jnp.float32)
            return carry
        lax.fori_loop(0, ZB, _fill_zero, 0)

        zbase = s * rpt_zero
        for i in range(nz_full):
            pltpu.sync_copy(zero_v, acc_s.at[pl.ds(zbase + i * ZB, ZB)])
        if nz_rem:
            pltpu.sync_copy(zero_v.at[pl.ds(0, nz_rem)],
                            acc_s.at[pl.ds(zbase + nz_full * ZB, nz_rem)])
        plsc.subcore_barrier()

        unit = s * NC + c if split_by_core else s
        first = unit * per_unit

        def issue_idx_load(b, p):
            ck0 = first + b * KB
            pltpu.async_copy(idxg_h.at[pl.ds(ck0, KB)], gidx_v.at[p], isem)
            pltpu.async_copy(idxs_h.at[pl.ds(ck0, KB)], sidx_v.at[p], isem)

        def wait_idx(p):
            pltpu.make_async_copy(idxg_h.at[pl.ds(0, KB)], gidx_v.at[p], isem).wait()
            pltpu.make_async_copy(idxs_h.at[pl.ds(0, KB)], sidx_v.at[p], isem).wait()

        def emit_block(b, p):
            wait_idx(p)
            issue_idx_load(b + 1, 1 - p)
            if core_shift:
                @pl.when(c == 1)
                def _():
                    off = jnp.full((LANE,), core_shift, jnp.int32)
                    for k in range(KB):
                        for g in range(CH // LANE):
                            sl = pl.ds(g * LANE, LANE)
                            gidx_v[p, k, sl] = gidx_v[p, k, sl] + off
            gathers = [
                pltpu.async_copy(table_h.at[gidx_v.at[p].at[j]], rows_v.at[j],
                                 gsem)
                for j in range(KB)
            ]
            adds = []
            for j in range(KB):
                gathers[j].wait()
                adds.append(
                    pltpu.async_copy(rows_v.at[j], acc_s.at[sidx_v.at[p].at[j]],
                                     ssem, add=True))
            for a in adds:
                a.wait()

        issue_idx_load(0, 0)

        def pair(i, carry):
            emit_block(2 * i, 0)
            emit_block(2 * i + 1, 1)
            return carry
        lax.fori_loop(0, nblk // 2, pair, 0)
        if nblk % 2:
            emit_block(nblk - 1, 0)
        wait_idx(nblk % 2)
        plsc.subcore_barrier()

        obase = s * rpt_out
        pltpu.sync_copy(acc_s.at[pl.ds(obase, rpt_out)],
                        out_h.at[c].at[pl.ds(obase, rpt_out)])

    return body(table, idxg, idxs)


def _combine_block(s_r, d_r, h_prev, bc_r, brp_r, h, r_num, bn):
    s0 = s_r[0]
    s1 = s_r[1]
    d = d_r[0] + d_r[1]
    hb = jnp.dot(h_prev, bc_r, preferred_element_type=jnp.float32)
    acc = jnp.concatenate([s0, s1], axis=-1)
    acc = acc + jnp.dot(d, brp_r, preferred_element_type=jnp.float32)
    for r in range(r_num):
        dr = d[:, r:r + 1]
        acc = acc + dr * hb[:, r * h:(r + 1) * h]
    return acc


def _tc_stage1(x, W0, b0, A1P):
    n, din = x.shape
    h = W0.shape[1]
    wide = A1P.shape[1]

    def body(x_r, w0_r, b0_r, ap_r, emb_r, g_r):
        e = jnp.dot(x_r[...], w0_r[...], preferred_element_type=jnp.float32)
        e = e + b0_r[...]
        emb_r[...] = e
        g_r[...] = jnp.dot(e, ap_r[...], preferred_element_type=jnp.float32)

    return pl.pallas_call(
        body,
        grid=(pl.cdiv(n, BN),),
        in_specs=[
            pl.BlockSpec((BN, din), lambda i: (i, 0)),
            pl.BlockSpec((din, h), lambda i: (0, 0)),
            pl.BlockSpec((1, h), lambda i: (0, 0)),
            pl.BlockSpec((h, wide), lambda i: (0, 0)),
        ],
        out_specs=[
            pl.BlockSpec((BN, h), lambda i: (i, 0)),
            pl.BlockSpec((BN, wide), lambda i: (i, 0)),
        ],
        out_shape=[
            jax.ShapeDtypeStruct((n, h), jnp.float32),
            jax.ShapeDtypeStruct((n, wide), jnp.float32),
        ],
    )(x, W0, b0.reshape(1, h), A1P)


def _tc_stage2(s1, deg, emb0, brp1, B1C, A2P):
    n = emb0.shape[0]
    h = B1C.shape[0]
    r_num = B1C.shape[1] // h
    wide = A2P.shape[1]

    def body(s_r, d_r, e0_r, br_r, bc_r, ap_r, g_r, e1_r):
        emb = _combine_block(s_r, d_r, e0_r[...], bc_r[...], br_r[...], h,
                             r_num, BN)
        e1_r[...] = emb
        g_r[...] = jnp.dot(emb, ap_r[...], preferred_element_type=jnp.float32)

    return pl.pallas_call(
        body,
        grid=(pl.cdiv(n, BN),),
        in_specs=[
            pl.BlockSpec((NC, BN, LANE), lambda i: (0, i, 0)),
            pl.BlockSpec((NC, BN, LANE), lambda i: (0, i, 0)),
            pl.BlockSpec((BN, h), lambda i: (i, 0)),
            pl.BlockSpec((LANE, h), lambda i: (0, 0)),
            pl.BlockSpec((h, wide), lambda i: (0, 0)),
            pl.BlockSpec((h, wide), lambda i: (0, 0)),
        ],
        out_specs=[
            pl.BlockSpec((BN, wide), lambda i: (i, 0)),
            pl.BlockSpec((BN, h), lambda i: (i, 0)),
        ],
        out_shape=[
            jax.ShapeDtypeStruct((n, wide), jnp.float32),
            jax.ShapeDtypeStruct((n, h), jnp.float32),
        ],
    )(s1, deg, emb0, brp1, B1C, A2P)


def _tc_stage3(s2, deg, emb1, brp2, B2C, emb0, Wf1, bf1, Wf2, bf2):
    n = emb0.shape[0]
    h = B2C.shape[0]
    r_num = B2C.shape[1] // h
    dmid = Wf1.shape[1]
    dout = Wf2.shape[1]

    def body(s_r, d_r, e1_r, br_r, bc_r, e0_r, wf1_r, bf1_r, wf2_r, bf2_r, o_r):
        emb2 = _combine_block(s_r, d_r, e1_r[...], bc_r[...], br_r[...], h,
                              r_num, BN)
        z = jnp.concatenate([emb2, e0_r[...]], axis=-1)
        z1 = jnp.dot(z, wf1_r[...], preferred_element_type=jnp.float32)
        z1 = z1 + bf1_r[...]
        z1 = jnp.where(z1 >= 0, z1, 0.01 * z1)
        o = jnp.dot(z1, wf2_r[...], preferred_element_type=jnp.float32)
        o = o + bf2_r[...]
        o_r[...] = jnp.where(o >= 0, o, 0.01 * o)

    return pl.pallas_call(
        body,
        grid=(pl.cdiv(n, BN),),
        in_specs=[
            pl.BlockSpec((NC, BN, LANE), lambda i: (0, i, 0)),
            pl.BlockSpec((NC, BN, LANE), lambda i: (0, i, 0)),
            pl.BlockSpec((BN, h), lambda i: (i, 0)),
            pl.BlockSpec((LANE, h), lambda i: (0, 0)),
            pl.BlockSpec((h, r_num * h), lambda i: (0, 0)),
            pl.BlockSpec((BN, h), lambda i: (i, 0)),
            pl.BlockSpec((2 * h, dmid), lambda i: (0, 0)),
            pl.BlockSpec((1, dmid), lambda i: (0, 0)),
            pl.BlockSpec((dmid, dout), lambda i: (0, 0)),
            pl.BlockSpec((1, dout), lambda i: (0, 0)),
        ],
        out_specs=[pl.BlockSpec((BN, dout), lambda i: (i, 0))],
        out_shape=[jax.ShapeDtypeStruct((n, dout), jnp.float32)],
    )(s2, deg, emb1, brp2, B2C, emb0, Wf1, bf1.reshape(1, dmid), Wf2,
      bf2.reshape(1, dout))[0]


def _a_perm(Wr, h):
    r_num = Wr.shape[0]
    a = Wr[:, :h, :]
    cols = [a[t][:, c * LANE:(c + 1) * LANE]
            for c in range(NC) for t in range(r_num)]
    return jnp.concatenate(cols, axis=1)


def _b_cat(Wr, h):
    b = Wr[:, h:, :]
    return jnp.concatenate([b[t] for t in range(Wr.shape[0])], axis=1)


def kernel(x, edge_index, edge_type, edge_time, pos,
           W0, b0, Wr1, br1, Wr2, br2, Wf1, bf1, Wf2, bf2):
    del edge_time, pos
    n = x.shape[0]
    h = W0.shape[1]
    r_num = Wr1.shape[0]
    e_num = edge_index.shape[1]
    assert h == 2 * LANE

    row = edge_index[0]
    col = edge_index[1]

    n_pad = ((n + NS * 8 - 1) // (NS * 8)) * NS * 8
    align = NW * KB
    nch = math.ceil(math.ceil(e_num / CH) / align) * align
    padn = nch * CH - e_num
    padn_a = padn + KB * CH
    gbase = row * (NC * r_num) + edge_type
    idxg = jnp.pad(gbase, (0, padn_a)).reshape(nch + KB, CH)
    idxs = jnp.pad(col, (0, padn_a), constant_values=n_pad).reshape(nch + KB, CH)
    rep = 4096
    idxt = jnp.pad(edge_type, (0, padn_a))
    idxt = idxt + 2 * r_num * (jnp.arange(idxt.shape[0], dtype=jnp.int32) & (rep - 1))
    idxt = idxt.reshape(nch + KB, CH)
    onehot = jnp.tile(jnp.eye(2 * r_num, LANE, dtype=jnp.float32), (rep, 1))

    A1P, B1C = _a_perm(Wr1, h), _b_cat(Wr1, h)
    A2P, B2C = _a_perm(Wr2, h), _b_cat(Wr2, h)

    brp1 = jnp.pad(br1, ((0, LANE - r_num), (0, 0)))
    brp2 = jnp.pad(br2, ((0, LANE - r_num), (0, 0)))

    deg = _sc_segment_pass(onehot, idxt, idxs, n_pad, split_by_core=True)
    emb0, g1 = _tc_stage1(x, W0, b0, A1P)
    s1 = _sc_segment_pass(g1.reshape(n * NC * r_num, LANE), idxg, idxs, n_pad,
                          split_by_core=False, core_shift=r_num)
    g2, emb1 = _tc_stage2(s1, deg, emb0, brp1, B1C, A2P)
    s2 = _sc_segment_pass(g2.reshape(n * NC * r_num, LANE), idxg, idxs, n_pad,
                          split_by_core=False, core_shift=r_num)
    return _tc_stage3(s2, deg, emb1, brp2, B2C, emb0, Wf1, bf1,
                      Wf2, bf2)
